# baseline (speedup 1.0000x reference)
import functools

import jax
import jax.numpy as jnp
from jax import lax
from jax.experimental import pallas as pl
from jax.experimental.pallas import tpu as pltpu
from jax.experimental.pallas import tpu_sc as plsc

_K = 16
_EPS = 1e-5


def _knn_body(pts_ref, cds_ref, idx_ref, *, rows, n):
    b = pl.program_id(0)
    pb = pts_ref[0]
    ca = cds_ref[0]
    dot = jnp.dot(pb.astype(jnp.bfloat16), ca.astype(jnp.bfloat16),
                  preferred_element_type=jnp.float32)
    r2b = jnp.sum(pb * pb, axis=1, keepdims=True)
    r2a = jnp.sum(ca * ca, axis=0)[None, :]
    dist = -2.0 * dot
    dist = dist + r2b
    dist = dist + r2a
    dist = jnp.maximum(dist, 1e-12)
    col = lax.broadcasted_iota(jnp.int32, (rows, n), 1)

    cols = []
    d = dist
    big = jnp.float32(jnp.inf)
    for t in range(_K + 1):
        m = jnp.min(d, axis=1, keepdims=True)
        cand = jnp.where(d == m, col, jnp.int32(n))
        it = jnp.min(cand, axis=1)
        if t > 0:
            cols.append(it)
        d = jnp.where(col == it[:, None], big, d)
    idx_ref[0] = jnp.stack(cols, axis=1) + b * n


def _knn(pts8, coords8):
    B, n, _ = pts8.shape
    rows = 512
    grid = (B, n // rows)
    return pl.pallas_call(
        functools.partial(_knn_body, rows=rows, n=n),
        grid=grid,
        in_specs=[
            pl.BlockSpec((1, rows, 8), lambda b, j: (b, j, 0)),
            pl.BlockSpec((1, 8, n), lambda b, j: (b, 0, 0)),
        ],
        out_specs=pl.BlockSpec((1, rows, _K), lambda b, j: (b, j, 0)),
        out_shape=jax.ShapeDtypeStruct((B, n, _K), jnp.int32),
    )(pts8, coords8)


def _dense1_body(ft_ref, wd_ref, wb_ref, u_ref, v_ref):
    ft = ft_ref[0]
    u_ref[0] = jnp.dot(ft, wd_ref[...], preferred_element_type=jnp.float32)
    v_ref[0] = jnp.dot(ft, wb_ref[...], preferred_element_type=jnp.float32)


def _dense1(featT, wdT, wbT):
    B, n, c = featT.shape
    co = wdT.shape[1]
    bn = 1024
    grid = (B, n // bn)
    return pl.pallas_call(
        _dense1_body,
        grid=grid,
        in_specs=[
            pl.BlockSpec((1, bn, c), lambda b, j: (b, j, 0)),
            pl.BlockSpec((c, co), lambda b, j: (0, 0)),
            pl.BlockSpec((c, co), lambda b, j: (0, 0)),
        ],
        out_specs=[
            pl.BlockSpec((1, bn, co), lambda b, j: (b, j, 0)),
            pl.BlockSpec((1, bn, co), lambda b, j: (b, j, 0)),
        ],
        out_shape=[
            jax.ShapeDtypeStruct((B, n, co), jnp.float32),
            jax.ShapeDtypeStruct((B, n, co), jnp.float32),
        ],
    )(featT, wdT, wbT)


def _sc_gather_reduce(vt_flat, idx_flat, c):
    total, _ = vt_flat.shape
    n_workers = 32
    pts_w = total // n_workers
    chunk = 8
    n_chunks = pts_w // chunk
    mesh = plsc.VectorSubcoreMesh(core_axis_name="c", subcore_axis_name="s",
                                  num_cores=2, num_subcores=16)

    def body(vt_hbm, idx_hbm, out_hbm,
             idx_v, rows_v, out_v, gsem0, gsem1, osem0, osem1):
        wid = lax.axis_index("s") * 2 + lax.axis_index("c")
        base0 = wid * pts_w
        gsems = (gsem0, gsem1)
        osems = (osem0, osem1)

        def issue(ci, sub):
            @pl.when(ci < n_chunks)
            def _():
                pbase = base0 + ci * chunk
                pltpu.sync_copy(idx_hbm.at[pl.ds(pbase * _K, chunk * _K)],
                                idx_v.at[sub])
                pltpu.async_copy(vt_hbm.at[idx_v.at[sub]], rows_v.at[sub],
                                 gsems[sub])

        def process(ci, sub):
            pbase = base0 + ci * chunk
            pltpu.make_async_copy(vt_hbm.at[idx_v.at[sub]], rows_v.at[sub],
                                  gsems[sub]).wait()

            @pl.when(ci >= 2)
            def _():
                pltpu.make_async_copy(out_v.at[sub],
                                      out_hbm.at[pl.ds(pbase, chunk)],
                                      osems[sub]).wait()

            def point_body(i, carry2):
                r0 = i * _K
                for g in range(c // 16):
                    sl = pl.ds(g * 16, 16)
                    v = rows_v[sub, r0, sl]
                    m = v
                    s = v
                    q = v * v
                    for k in range(1, _K):
                        v = rows_v[sub, r0 + k, sl]
                        m = jnp.maximum(m, v)
                        s = s + v
                        q = q + v * v
                    out_v[sub, i, 0, sl] = m
                    out_v[sub, i, 1, sl] = s
                    out_v[sub, i, 2, sl] = q
                return carry2

            lax.fori_loop(0, chunk, point_body, 0, unroll=False)
            issue(ci + 2, sub)
            pltpu.async_copy(out_v.at[sub], out_hbm.at[pl.ds(pbase, chunk)],
                             osems[sub])

        issue(0, 0)
        issue(1, 1)

        def pair_body(pi, carry):
            process(2 * pi, 0)
            process(2 * pi + 1, 1)
            return carry

        lax.fori_loop(0, n_chunks // 2, pair_body, 0, unroll=False)
        for sub in range(2):
            last = base0 + (n_chunks - 2 + sub) * chunk
            pltpu.make_async_copy(out_v.at[sub],
                                  out_hbm.at[pl.ds(last, chunk)],
                                  osems[sub]).wait()

    fn = pl.kernel(
        body,
        out_type=jax.ShapeDtypeStruct((total, 3, c), jnp.float32),
        mesh=mesh,
        scratch_types=[
            pltpu.VMEM((2, chunk * _K), jnp.int32),
            pltpu.VMEM((2, chunk * _K, c), jnp.float32),
            pltpu.VMEM((2, chunk, 3, c), jnp.float32),
            pltpu.SemaphoreType.DMA,
            pltpu.SemaphoreType.DMA,
            pltpu.SemaphoreType.DMA,
            pltpu.SemaphoreType.DMA,
        ],
    )
    return fn(vt_flat, idx_flat)


def _stats_body(u_ref, s_ref, q_ref, out_ref):
    j = pl.program_id(1)
    u = u_ref[0]
    s = s_ref[0]
    q = q_ref[0]
    kf = jnp.float32(_K)
    p0 = jnp.sum(kf * u + s, axis=0)
    p1 = jnp.sum(kf * (u * u) + 2.0 * u * s + q, axis=0)
    part = jnp.stack([p0, p1], axis=0)

    @pl.when(j == 0)
    def _():
        out_ref[0] = jnp.zeros_like(out_ref[0])

    out_ref[0] += part


def _stats(u, packed):
    B, n, c = u.shape
    bn = 1024
    grid = (B, n // bn)
    spec = pl.BlockSpec((1, bn, c), lambda b, j: (b, j, 0))
    sspec = pl.BlockSpec((1, bn, c), lambda b, j: (b, j, 1))
    qspec = pl.BlockSpec((1, bn, c), lambda b, j: (b, j, 2))
    return pl.pallas_call(
        _stats_body,
        grid=grid,
        in_specs=[spec, sspec, qspec],
        out_specs=pl.BlockSpec((1, 2, c), lambda b, j: (b, 0, 0)),
        out_shape=jax.ShapeDtypeStruct((B, 2, c), jnp.float32),
    )(u, packed, packed)


def _norm_from_stats(st_ref, count):
    mean = st_ref[0, 0, :] / count
    esq = st_ref[0, 1, :] / count
    var = esq - mean * mean
    inv = lax.rsqrt(var + _EPS)
    return mean, inv


def _lrelu(x):
    return jnp.where(x >= 0, x, 0.2 * x)


def _norm_dense_body(u_ref, m_ref, st_ref, wd_ref, wb_ref,
                     x_ref, u2_ref, v2_ref, *, count):
    mean, inv = _norm_from_stats(st_ref, count)
    ymax = u_ref[0] + m_ref[0]
    x = _lrelu((ymax - mean[None, :]) * inv[None, :])
    x_ref[0] = x
    u2_ref[0] = jnp.dot(x, wd_ref[...], preferred_element_type=jnp.float32)
    v2_ref[0] = jnp.dot(x, wb_ref[...], preferred_element_type=jnp.float32)


def _norm_dense(u, m, st, wdT, wbT, count):
    B, n, c = u.shape
    co = wdT.shape[1]
    bn = 1024
    grid = (B, n // bn)
    spec = pl.BlockSpec((1, bn, c), lambda b, j: (b, j, 0))
    mspec = pl.BlockSpec((1, bn, c), lambda b, j: (b, j, 0))
    wspec = pl.BlockSpec((c, co), lambda b, j: (0, 0))
    ospec = pl.BlockSpec((1, bn, co), lambda b, j: (b, j, 0))
    return pl.pallas_call(
        functools.partial(_norm_dense_body, count=count),
        grid=grid,
        in_specs=[spec, mspec,
                  pl.BlockSpec((1, 2, c), lambda b, j: (b, 0, 0)),
                  wspec, wspec],
        out_specs=[spec, ospec, ospec],
        out_shape=[
            jax.ShapeDtypeStruct((B, n, c), jnp.float32),
            jax.ShapeDtypeStruct((B, n, co), jnp.float32),
            jax.ShapeDtypeStruct((B, n, co), jnp.float32),
        ],
    )(u, m, st, wdT, wbT)


def _norm2_y3_body(u_ref, m_ref, st_ref, ft_ref, x1_ref, w3_ref,
                   y3_ref, st3_ref, *, count, c1):
    j = pl.program_id(1)
    mean, inv = _norm_from_stats(st_ref, count)
    ymax = u_ref[0] + m_ref[0]
    x2 = _lrelu((ymax - mean[None, :]) * inv[None, :])
    w3 = w3_ref[...]
    y3 = (jnp.dot(ft_ref[0], w3[:c1], preferred_element_type=jnp.float32)
          + jnp.dot(x1_ref[0], w3[c1:2 * c1], preferred_element_type=jnp.float32)
          + jnp.dot(x2, w3[2 * c1:], preferred_element_type=jnp.float32))
    y3_ref[0] = y3
    part = jnp.stack([jnp.sum(y3, axis=0), jnp.sum(y3 * y3, axis=0)], axis=0)

    @pl.when(j == 0)
    def _():
        st3_ref[0] = jnp.zeros_like(st3_ref[0])

    st3_ref[0] += part


def _norm2_y3(u2, m2, st2, featT, x1, w3T, count):
    B, n, c2 = u2.shape
    c1 = featT.shape[2]
    bn = 1024
    grid = (B, n // bn)
    spec2 = pl.BlockSpec((1, bn, c2), lambda b, j: (b, j, 0))
    mspec2 = pl.BlockSpec((1, bn, c2), lambda b, j: (b, j, 0))
    spec1 = pl.BlockSpec((1, bn, c1), lambda b, j: (b, j, 0))
    return pl.pallas_call(
        functools.partial(_norm2_y3_body, count=count, c1=c1),
        grid=grid,
        in_specs=[spec2, mspec2,
                  pl.BlockSpec((1, 2, c2), lambda b, j: (b, 0, 0)),
                  spec1, spec1,
                  pl.BlockSpec((4 * c1, c1), lambda b, j: (0, 0))],
        out_specs=[spec1, pl.BlockSpec((1, 2, c1), lambda b, j: (b, 0, 0))],
        out_shape=[
            jax.ShapeDtypeStruct((B, n, c1), jnp.float32),
            jax.ShapeDtypeStruct((B, 2, c1), jnp.float32),
        ],
    )(u2, m2, st2, featT, x1, w3T)


def _final_body(y_ref, st_ref, out_ref, *, count):
    mean, inv = _norm_from_stats(st_ref, count)
    out_ref[0] = _lrelu((y_ref[0] - mean[None, :]) * inv[None, :])


def _final(y3, st3, count):
    B, n, c = y3.shape
    bn = 1024
    grid = (B, n // bn)
    spec = pl.BlockSpec((1, bn, c), lambda b, j: (b, j, 0))
    return pl.pallas_call(
        functools.partial(_final_body, count=count),
        grid=grid,
        in_specs=[spec, pl.BlockSpec((1, 2, c), lambda b, j: (b, 0, 0))],
        out_specs=spec,
        out_shape=jax.ShapeDtypeStruct((B, n, c), jnp.float32),
    )(y3, st3)


def kernel(coords, features, W1, W2, W3):
    B, C, N = features.shape
    featT = features.transpose(0, 2, 1)
    coords8 = jnp.concatenate(
        [coords, jnp.zeros((B, 5, N), coords.dtype)], axis=1)
    pts8 = coords8.transpose(0, 2, 1)

    idx = _knn(pts8, coords8)
    idx_flat = idx.reshape(B * N * _K)

    w1a, w1b = W1[:, :C], W1[:, C:]
    u1, v1 = _dense1(featT, (w1a - w1b).T, w1b.T)
    pk1 = _sc_gather_reduce(v1.reshape(B * N, C), idx_flat, C)
    pk1 = pk1.reshape(B, N, 3 * C)
    st1 = _stats(u1, pk1)

    c2 = W2.shape[0]
    w2a, w2b = W2[:, :C], W2[:, C:]
    x1, u2, v2 = _norm_dense(u1, pk1, st1, (w2a - w2b).T, w2b.T,
                             float(N * _K))
    pk2 = _sc_gather_reduce(v2.reshape(B * N, c2), idx_flat, c2)
    pk2 = pk2.reshape(B, N, 3 * c2)
    st2 = _stats(u2, pk2)

    y3, st3 = _norm2_y3(u2, pk2, st2, featT, x1, W3.T, float(N * _K))
    out = _final(y3, st3, float(N))
    return out.transpose(0, 2, 1)


# speedup vs baseline: 1.1488x; 1.1488x over previous
import functools

import jax
import jax.numpy as jnp
from jax import lax
from jax.experimental import pallas as pl
from jax.experimental.pallas import tpu as pltpu
from jax.experimental.pallas import tpu_sc as plsc

_K = 16
_EPS = 1e-5


def _knn_body(pts_ref, cds_ref, idx_ref, *, rows, n):
    b = pl.program_id(0)
    pb = pts_ref[0]
    ca = cds_ref[0]
    dot = jnp.dot(pb.astype(jnp.bfloat16), ca.astype(jnp.bfloat16),
                  preferred_element_type=jnp.float32)
    r2b = jnp.sum(pb * pb, axis=1, keepdims=True)
    r2a = jnp.sum(ca * ca, axis=0)[None, :]
    dist = -2.0 * dot
    dist = dist + r2b
    dist = dist + r2a
    dist = jnp.maximum(dist, 1e-12)
    col = lax.broadcasted_iota(jnp.int32, (rows, n), 1)

    cols = []
    d = dist
    big = jnp.float32(jnp.inf)
    for t in range(_K + 1):
        m = jnp.min(d, axis=1, keepdims=True)
        cand = jnp.where(d == m, col, jnp.int32(n))
        it = jnp.min(cand, axis=1)
        if t > 0:
            cols.append(it)
        d = jnp.where(col == it[:, None], big, d)
    idx_ref[0] = jnp.stack(cols, axis=1) + b * n


def _knn(pts8, coords8):
    B, n, _ = pts8.shape
    rows = 256
    grid = (B, n // rows)
    return pl.pallas_call(
        functools.partial(_knn_body, rows=rows, n=n),
        grid=grid,
        in_specs=[
            pl.BlockSpec((1, rows, 8), lambda b, j: (b, j, 0)),
            pl.BlockSpec((1, 8, n), lambda b, j: (b, 0, 0)),
        ],
        out_specs=pl.BlockSpec((1, rows, _K), lambda b, j: (b, j, 0)),
        out_shape=jax.ShapeDtypeStruct((B, n, _K), jnp.int32),
    )(pts8, coords8)


def _dense1_body(ft_ref, wd_ref, wb_ref, u_ref, v_ref):
    ft = ft_ref[0]
    u_ref[0] = jnp.dot(ft, wd_ref[...], preferred_element_type=jnp.float32)
    v_ref[0] = jnp.dot(ft, wb_ref[...], preferred_element_type=jnp.float32)


def _dense1(featT, wdT, wbT):
    B, n, c = featT.shape
    co = wdT.shape[1]
    bn = 1024
    grid = (B, n // bn)
    return pl.pallas_call(
        _dense1_body,
        grid=grid,
        in_specs=[
            pl.BlockSpec((1, bn, c), lambda b, j: (b, j, 0)),
            pl.BlockSpec((c, co), lambda b, j: (0, 0)),
            pl.BlockSpec((c, co), lambda b, j: (0, 0)),
        ],
        out_specs=[
            pl.BlockSpec((1, bn, co), lambda b, j: (b, j, 0)),
            pl.BlockSpec((1, bn, co), lambda b, j: (b, j, 0)),
        ],
        out_shape=[
            jax.ShapeDtypeStruct((B, n, co), jnp.float32),
            jax.ShapeDtypeStruct((B, n, co), jnp.float32),
        ],
    )(featT, wdT, wbT)


def _sc_gather_reduce(vt_flat, idx_flat, c):
    total, _ = vt_flat.shape
    n_workers = 32
    pts_w = total // n_workers
    chunk = 8
    n_chunks = pts_w // chunk
    mesh = plsc.VectorSubcoreMesh(core_axis_name="c", subcore_axis_name="s",
                                  num_cores=2, num_subcores=16)

    def body(vt_hbm, idx_hbm, out_hbm,
             idx_v, rows_v, out_v, gsem0, gsem1, osem0, osem1):
        wid = lax.axis_index("s") * 2 + lax.axis_index("c")
        base0 = wid * pts_w
        gsems = (gsem0, gsem1)
        osems = (osem0, osem1)

        def issue(ci, sub):
            @pl.when(ci < n_chunks)
            def _():
                pbase = base0 + ci * chunk
                pltpu.sync_copy(idx_hbm.at[pl.ds(pbase * _K, chunk * _K)],
                                idx_v.at[sub])
                pltpu.async_copy(vt_hbm.at[idx_v.at[sub]], rows_v.at[sub],
                                 gsems[sub])

        def process(ci, sub):
            pbase = base0 + ci * chunk
            pltpu.make_async_copy(vt_hbm.at[idx_v.at[sub]], rows_v.at[sub],
                                  gsems[sub]).wait()

            @pl.when(ci >= 2)
            def _():
                pltpu.make_async_copy(out_v.at[sub],
                                      out_hbm.at[pl.ds(pbase, chunk)],
                                      osems[sub]).wait()

            def point_body(i, carry2):
                r0 = i * _K
                for g in range(c // 16):
                    sl = pl.ds(g * 16, 16)
                    v = rows_v[sub, r0, sl]
                    m = v
                    s = v
                    q = v * v
                    for k in range(1, _K):
                        v = rows_v[sub, r0 + k, sl]
                        m = jnp.maximum(m, v)
                        s = s + v
                        q = q + v * v
                    out_v[sub, i, 0, sl] = m
                    out_v[sub, i, 1, sl] = s
                    out_v[sub, i, 2, sl] = q
                return carry2

            lax.fori_loop(0, chunk, point_body, 0, unroll=False)
            issue(ci + 2, sub)
            pltpu.async_copy(out_v.at[sub], out_hbm.at[pl.ds(pbase, chunk)],
                             osems[sub])

        issue(0, 0)
        issue(1, 1)

        def pair_body(pi, carry):
            process(2 * pi, 0)
            process(2 * pi + 1, 1)
            return carry

        lax.fori_loop(0, n_chunks // 2, pair_body, 0, unroll=False)
        for sub in range(2):
            last = base0 + (n_chunks - 2 + sub) * chunk
            pltpu.make_async_copy(out_v.at[sub],
                                  out_hbm.at[pl.ds(last, chunk)],
                                  osems[sub]).wait()

    fn = pl.kernel(
        body,
        out_type=jax.ShapeDtypeStruct((total, 3, c), jnp.float32),
        mesh=mesh,
        scratch_types=[
            pltpu.VMEM((2, chunk * _K), jnp.int32),
            pltpu.VMEM((2, chunk * _K, c), jnp.float32),
            pltpu.VMEM((2, chunk, 3, c), jnp.float32),
            pltpu.SemaphoreType.DMA,
            pltpu.SemaphoreType.DMA,
            pltpu.SemaphoreType.DMA,
            pltpu.SemaphoreType.DMA,
        ],
    )
    return fn(vt_flat, idx_flat)


def _stats_body(u_ref, s_ref, q_ref, out_ref):
    j = pl.program_id(1)
    u = u_ref[0]
    s = s_ref[0]
    q = q_ref[0]
    kf = jnp.float32(_K)
    p0 = jnp.sum(kf * u + s, axis=0)
    p1 = jnp.sum(kf * (u * u) + 2.0 * u * s + q, axis=0)
    part = jnp.stack([p0, p1], axis=0)

    @pl.when(j == 0)
    def _():
        out_ref[0] = jnp.zeros_like(out_ref[0])

    out_ref[0] += part


def _stats(u, packed):
    B, n, c = u.shape
    bn = 1024
    grid = (B, n // bn)
    spec = pl.BlockSpec((1, bn, c), lambda b, j: (b, j, 0))
    sspec = pl.BlockSpec((1, bn, c), lambda b, j: (b, j, 1))
    qspec = pl.BlockSpec((1, bn, c), lambda b, j: (b, j, 2))
    return pl.pallas_call(
        _stats_body,
        grid=grid,
        in_specs=[spec, sspec, qspec],
        out_specs=pl.BlockSpec((1, 2, c), lambda b, j: (b, 0, 0)),
        out_shape=jax.ShapeDtypeStruct((B, 2, c), jnp.float32),
    )(u, packed, packed)


def _norm_from_stats(st_ref, count):
    mean = st_ref[0, 0, :] / count
    esq = st_ref[0, 1, :] / count
    var = esq - mean * mean
    inv = lax.rsqrt(var + _EPS)
    return mean, inv


def _lrelu(x):
    return jnp.where(x >= 0, x, 0.2 * x)


def _norm_dense_body(u_ref, m_ref, st_ref, wd_ref, wb_ref,
                     x_ref, u2_ref, v2_ref, *, count):
    mean, inv = _norm_from_stats(st_ref, count)
    ymax = u_ref[0] + m_ref[0]
    x = _lrelu((ymax - mean[None, :]) * inv[None, :])
    x_ref[0] = x
    u2_ref[0] = jnp.dot(x, wd_ref[...], preferred_element_type=jnp.float32)
    v2_ref[0] = jnp.dot(x, wb_ref[...], preferred_element_type=jnp.float32)


def _norm_dense(u, m, st, wdT, wbT, count):
    B, n, c = u.shape
    co = wdT.shape[1]
    bn = 1024
    grid = (B, n // bn)
    spec = pl.BlockSpec((1, bn, c), lambda b, j: (b, j, 0))
    mspec = pl.BlockSpec((1, bn, c), lambda b, j: (b, j, 0))
    wspec = pl.BlockSpec((c, co), lambda b, j: (0, 0))
    ospec = pl.BlockSpec((1, bn, co), lambda b, j: (b, j, 0))
    return pl.pallas_call(
        functools.partial(_norm_dense_body, count=count),
        grid=grid,
        in_specs=[spec, mspec,
                  pl.BlockSpec((1, 2, c), lambda b, j: (b, 0, 0)),
                  wspec, wspec],
        out_specs=[spec, ospec, ospec],
        out_shape=[
            jax.ShapeDtypeStruct((B, n, c), jnp.float32),
            jax.ShapeDtypeStruct((B, n, co), jnp.float32),
            jax.ShapeDtypeStruct((B, n, co), jnp.float32),
        ],
    )(u, m, st, wdT, wbT)


def _norm2_y3_body(u_ref, m_ref, st_ref, ft_ref, x1_ref, w3_ref,
                   y3_ref, st3_ref, *, count, c1):
    j = pl.program_id(1)
    mean, inv = _norm_from_stats(st_ref, count)
    ymax = u_ref[0] + m_ref[0]
    x2 = _lrelu((ymax - mean[None, :]) * inv[None, :])
    w3 = w3_ref[...]
    y3 = (jnp.dot(ft_ref[0], w3[:c1], preferred_element_type=jnp.float32)
          + jnp.dot(x1_ref[0], w3[c1:2 * c1], preferred_element_type=jnp.float32)
          + jnp.dot(x2, w3[2 * c1:], preferred_element_type=jnp.float32))
    y3_ref[0] = y3
    part = jnp.stack([jnp.sum(y3, axis=0), jnp.sum(y3 * y3, axis=0)], axis=0)

    @pl.when(j == 0)
    def _():
        st3_ref[0] = jnp.zeros_like(st3_ref[0])

    st3_ref[0] += part


def _norm2_y3(u2, m2, st2, featT, x1, w3T, count):
    B, n, c2 = u2.shape
    c1 = featT.shape[2]
    bn = 1024
    grid = (B, n // bn)
    spec2 = pl.BlockSpec((1, bn, c2), lambda b, j: (b, j, 0))
    mspec2 = pl.BlockSpec((1, bn, c2), lambda b, j: (b, j, 0))
    spec1 = pl.BlockSpec((1, bn, c1), lambda b, j: (b, j, 0))
    return pl.pallas_call(
        functools.partial(_norm2_y3_body, count=count, c1=c1),
        grid=grid,
        in_specs=[spec2, mspec2,
                  pl.BlockSpec((1, 2, c2), lambda b, j: (b, 0, 0)),
                  spec1, spec1,
                  pl.BlockSpec((4 * c1, c1), lambda b, j: (0, 0))],
        out_specs=[spec1, pl.BlockSpec((1, 2, c1), lambda b, j: (b, 0, 0))],
        out_shape=[
            jax.ShapeDtypeStruct((B, n, c1), jnp.float32),
            jax.ShapeDtypeStruct((B, 2, c1), jnp.float32),
        ],
    )(u2, m2, st2, featT, x1, w3T)


def _final_body(y_ref, st_ref, out_ref, *, count):
    mean, inv = _norm_from_stats(st_ref, count)
    out_ref[0] = _lrelu((y_ref[0] - mean[None, :]) * inv[None, :])


def _final(y3, st3, count):
    B, n, c = y3.shape
    bn = 1024
    grid = (B, n // bn)
    spec = pl.BlockSpec((1, bn, c), lambda b, j: (b, j, 0))
    return pl.pallas_call(
        functools.partial(_final_body, count=count),
        grid=grid,
        in_specs=[spec, pl.BlockSpec((1, 2, c), lambda b, j: (b, 0, 0))],
        out_specs=spec,
        out_shape=jax.ShapeDtypeStruct((B, n, c), jnp.float32),
    )(y3, st3)


def kernel(coords, features, W1, W2, W3):
    B, C, N = features.shape
    featT = features.transpose(0, 2, 1)
    coords8 = jnp.concatenate(
        [coords, jnp.zeros((B, 5, N), coords.dtype)], axis=1)
    pts8 = coords8.transpose(0, 2, 1)

    idx = _knn(pts8, coords8)
    idx_flat = idx.reshape(B * N * _K)

    w1a, w1b = W1[:, :C], W1[:, C:]
    u1, v1 = _dense1(featT, (w1a - w1b).T, w1b.T)
    pk1 = _sc_gather_reduce(v1.reshape(B * N, C), idx_flat, C)
    pk1 = pk1.reshape(B, N, 3 * C)
    st1 = _stats(u1, pk1)

    c2 = W2.shape[0]
    w2a, w2b = W2[:, :C], W2[:, C:]
    x1, u2, v2 = _norm_dense(u1, pk1, st1, (w2a - w2b).T, w2b.T,
                             float(N * _K))
    pk2 = _sc_gather_reduce(v2.reshape(B * N, c2), idx_flat, c2)
    pk2 = pk2.reshape(B, N, 3 * c2)
    st2 = _stats(u2, pk2)

    y3, st3 = _norm2_y3(u2, pk2, st2, featT, x1, W3.T, float(N * _K))
    out = _final(y3, st3, float(N))
    return out.transpose(0, 2, 1)
